# Initial kernel scaffold; baseline (speedup 1.0000x reference)
import jax
import jax.numpy as jnp
from jax.experimental import pallas as pl


def kernel(x, edge_index, W1, b1, W3, b3):
    raise NotImplementedError("write your pallas kernel here")


# speedup vs baseline: 375.1148x; 375.1148x over previous
import jax
import jax.numpy as jnp
from jax import lax
from jax.experimental import pallas as pl
from jax.experimental.pallas import tpu as pltpu
from jax.experimental.pallas import tpu_sc as plsc

N = 100000
E = 1600000
B = 2
C = 4
LAYERS = 3

NS = 16
NPAD = 100352
NODE_T = NPAD // NS
ROWS = 12544
EPAD = ROWS * 128
ROWS_T = ROWS // NS
CHR = 28
CH = CHR * 128
NCHUNK = ROWS_T // CHR
GR = 1568
NGCH = NODE_T // GR

_SQRT_2_OVER_PI = 0.7978845608028654
_GELU_COEF = 0.044715


def _gelu(z):
  u = _SQRT_2_OVER_PI * (z + _GELU_COEF * z * z * z)
  a = jnp.abs(u)
  t = 1.0 - 2.0 / (jnp.exp(2.0 * a) + 1.0)
  t = jnp.where(u < 0.0, -t, t)
  return 0.5 * z * (1.0 + t)


def _body(
    xp,
    srcf,
    dstf,
    w1t,
    w3t,
    b1t,
    b3p,
    out,
    xe_hbm,
    x_tab,
    h0_tab,
    h1_tab,
    h2_tab,
    h3_tab,
    o_tab,
    dbuf,
    sbuf,
    xebuf,
    xebuf2,
    w0buf,
    w1buf,
    w2buf,
    w3buf,
    v0buf,
    v1buf,
    v2buf,
    v3buf,
    b3buf,
    hbuf,
    b1buf,
    zb,
    semg,
    sems,
    seml,
):
  b = lax.axis_index("c")
  s = lax.axis_index("s")
  node0 = s * NODE_T
  row0 = s * ROWS_T
  h_tabs = (h0_tab, h1_tab, h2_tab, h3_tab)
  w_bufs = (w0buf, w1buf, w2buf, w3buf)
  v_bufs = (v0buf, v1buf, v2buf, v3buf)

  @pl.loop(0, GR // 224)
  def _zinit(k):
    zb[pl.ds(k * 224, 224)] = jnp.zeros((224,), jnp.float32)

  for t in range(NGCH):
    r = node0 + GR * t
    pltpu.sync_copy(xp.at[pl.ds(b * NPAD + r, GR)], hbuf)
    pltpu.sync_copy(hbuf, x_tab.at[pl.ds(r, GR)])
    pltpu.sync_copy(zb, o_tab.at[pl.ds(r, GR)])
  plsc.subcore_barrier()

  for l in range(LAYERS):
    for c in range(C):
      for t in range(NGCH):
        pltpu.sync_copy(zb, h_tabs[c].at[pl.ds(node0 + GR * t, GR)])
    plsc.subcore_barrier()

    @pl.loop(0, NCHUNK)
    def _phase_a(g):
      e = (row0 + g * CHR) * 128
      loads = [pltpu.async_copy(dstf.at[pl.ds(e, CH)], dbuf, seml)]
      for c in range(C):
        loads.append(
            pltpu.async_copy(w1t.at[pl.ds(c * EPAD + e, CH)], w_bufs[c], seml)
        )
      if l == 0:
        pltpu.sync_copy(srcf.at[pl.ds(e, CH)], sbuf)
        g_x = pltpu.async_copy(x_tab.at[sbuf], xebuf, semg)
        for d in loads:
          d.wait()
        g_x.wait()
        pltpu.sync_copy(xebuf, xe_hbm.at[pl.ds(b * EPAD + e, CH)])
      else:
        loads.append(
            pltpu.async_copy(xe_hbm.at[pl.ds(b * EPAD + e, CH)], xebuf, seml)
        )
        for d in loads:
          d.wait()

      @pl.loop(0, CH // 128)
      def _vals(k):
        d = pl.ds(k * 128, 128)
        xe = xebuf[d]
        for c in range(C):
          v_bufs[c][d] = w_bufs[c][d] * xe

      descs = [
          pltpu.async_copy(v_bufs[c], h_tabs[c].at[dbuf], sems, add=True)
          for c in range(C)
      ]
      for d in descs:
        d.wait()

    plsc.subcore_barrier()

    for c in range(C):
      @pl.loop(0, NGCH)
      def _phase_b(t):
        r = node0 + GR * t
        pltpu.sync_copy(h_tabs[c].at[pl.ds(r, GR)], hbuf)
        pltpu.sync_copy(b1t.at[pl.ds(c * NPAD + r, GR)], b1buf)

        @pl.loop(0, GR // 224)
        def _act(k):
          d = pl.ds(k * 224, 224)
          hbuf[d] = _gelu(hbuf[d] + b1buf[d])

        pltpu.sync_copy(hbuf, h_tabs[c].at[pl.ds(r, GR)])

    plsc.subcore_barrier()

    @pl.loop(0, NCHUNK)
    def _phase_c(g):
      e = (row0 + g * CHR) * 128
      pltpu.sync_copy(srcf.at[pl.ds(e, CH)], sbuf)
      gathers = [
          pltpu.async_copy(h_tabs[c].at[sbuf], v_bufs[c], semg)
          for c in range(C)
      ]
      loads = []
      for c in range(C):
        loads.append(
            pltpu.async_copy(w3t.at[pl.ds(c * EPAD + e, CH)], w_bufs[c], seml)
        )
      loads.append(pltpu.async_copy(b3p.at[pl.ds(e, CH)], b3buf, seml))
      loads.append(
          pltpu.async_copy(xe_hbm.at[pl.ds(b * EPAD + e, CH)], xebuf, seml)
      )
      if l == LAYERS - 1:
        loads.append(pltpu.async_copy(dstf.at[pl.ds(e, CH)], dbuf, seml))
      for d in loads:
        d.wait()
      for d in gathers:
        d.wait()

      @pl.loop(0, CH // 128)
      def _red(k):
        d = pl.ds(k * 128, 128)
        oe = b3buf[d]
        for c in range(C):
          oe = oe + v_bufs[c][d] * w_bufs[c][d]
        xebuf2[d] = xebuf[d] + oe / 3.0

      pltpu.sync_copy(xebuf2, xe_hbm.at[pl.ds(b * EPAD + e, CH)])
      if l == LAYERS - 1:
        pltpu.async_copy(xebuf2, o_tab.at[dbuf], sems, add=True).wait()

    plsc.subcore_barrier()

  for t in range(NGCH):
    r = node0 + GR * t
    pltpu.sync_copy(o_tab.at[pl.ds(r, GR)], hbuf)
    pltpu.sync_copy(hbuf, out.at[pl.ds(b * NPAD + r, GR)])


@jax.jit
def kernel(x, edge_index, W1, b1, W3, b3):
  assert x.shape == (B, N) and edge_index.shape == (2, E)

  xp = jnp.pad(x, ((0, 0), (0, NPAD - N))).reshape(B * NPAD)
  ei = jnp.pad(edge_index, ((0, 0), (0, EPAD - E)), constant_values=N)
  srcf = ei[0]
  dstf = ei[1]
  w1t = jnp.pad(W1, ((0, EPAD - E), (0, 0))).T.reshape(C * EPAD)
  w3t = jnp.pad(W3, ((0, EPAD - E), (0, 0))).T.reshape(C * EPAD)
  b1t = jnp.pad(b1, ((0, NPAD - N), (0, 0))).T.reshape(C * NPAD)
  b3p = jnp.pad(b3, (0, EPAD - E))

  mesh = plsc.VectorSubcoreMesh(core_axis_name="c", subcore_axis_name="s")
  f = pl.kernel(
      _body,
      out_type=(
          jax.ShapeDtypeStruct((B * NPAD,), jnp.float32),
          jax.ShapeDtypeStruct((B * EPAD,), jnp.float32),
      ),
      mesh=mesh,
      scratch_types=[
          pltpu.VMEM_SHARED((NPAD,), jnp.float32),
          pltpu.VMEM_SHARED((NPAD,), jnp.float32),
          pltpu.VMEM_SHARED((NPAD,), jnp.float32),
          pltpu.VMEM_SHARED((NPAD,), jnp.float32),
          pltpu.VMEM_SHARED((NPAD,), jnp.float32),
          pltpu.VMEM_SHARED((NPAD,), jnp.float32),
          pltpu.VMEM((CH,), jnp.int32),
          pltpu.VMEM((CH,), jnp.int32),
          pltpu.VMEM((CH,), jnp.float32),
          pltpu.VMEM((CH,), jnp.float32),
          pltpu.VMEM((CH,), jnp.float32),
          pltpu.VMEM((CH,), jnp.float32),
          pltpu.VMEM((CH,), jnp.float32),
          pltpu.VMEM((CH,), jnp.float32),
          pltpu.VMEM((CH,), jnp.float32),
          pltpu.VMEM((CH,), jnp.float32),
          pltpu.VMEM((CH,), jnp.float32),
          pltpu.VMEM((CH,), jnp.float32),
          pltpu.VMEM((CH,), jnp.float32),
          pltpu.VMEM((GR,), jnp.float32),
          pltpu.VMEM((GR,), jnp.float32),
          pltpu.VMEM((GR,), jnp.float32),
          pltpu.SemaphoreType.DMA,
          pltpu.SemaphoreType.DMA,
          pltpu.SemaphoreType.DMA,
      ],
  )
  node_out, _ = f(xp, srcf, dstf, w1t, w3t, b1t, b3p)
  return node_out.reshape(B, NPAD)[:, :N]
